# baseline (speedup 1.0000x reference)
import functools
import math

import jax
import jax.numpy as jnp
from jax import lax
from jax.experimental import pallas as pl
from jax.experimental.pallas import tpu as pltpu
from jax.experimental.pallas import tpu_sc as plsc

_D = 128
_SCALE = math.sqrt(float(_D))
_NC = 2
_NS = 16
_NW = _NC * _NS
_BPC = 2
_NBUF = 4


def _make_lookup(B: int, L: int):
    assert B % (_NW * _BPC * _NBUF) == 0
    b_per_w = B // _NW
    n_chunks = b_per_w // _BPC
    n_groups = n_chunks // _NBUF
    assert n_groups >= 3
    rows = _BPC * L
    mesh = plsc.VectorSubcoreMesh(
        core_axis_name="c", subcore_axis_name="s", num_cores=_NC, num_subcores=_NS
    )

    @functools.partial(
        pl.kernel,
        mesh=mesh,
        out_type=jax.ShapeDtypeStruct((B, L, _D), jnp.float32),
        scratch_types=(
            [pltpu.VMEM((n_chunks, rows), jnp.int32)]
            + [pltpu.VMEM((rows, _D), jnp.float32) for _ in range(2 * _NBUF)]
            + [pltpu.SemaphoreType.DMA for _ in range(2 * _NBUF)]
        ),
        compiler_params=pltpu.CompilerParams(use_tc_tiling_on_sc=True),
    )
    def lookup(idx_hbm, table_hbm, out_hbm, idx_v, *rest):
        ins = rest[:_NBUF]
        outs = rest[_NBUF:2 * _NBUF]
        gsems = rest[2 * _NBUF:3 * _NBUF]
        ssems = rest[3 * _NBUF:4 * _NBUF]
        wid = lax.axis_index("s") * _NC + lax.axis_index("c")
        base = wid * b_per_w
        pltpu.sync_copy(idx_hbm.at[wid], idx_v)

        def gather(c, s):
            return pltpu.make_async_copy(
                table_hbm.at[idx_v.at[c]], ins[s], gsems[s]
            )

        def stores(c, s):
            return [
                pltpu.make_async_copy(
                    outs[s].at[pl.ds(i * L, L)],
                    out_hbm.at[base + c * _BPC + i],
                    ssems[s],
                )
                for i in range(_BPC)
            ]

        def start_stores(c, s):
            for cp in stores(c, s):
                cp.start()

        def wait_stores(c, s):
            for cp in stores(c, s):
                cp.wait()

        def scale(s):
            def scale_row(r, carry):
                for j in range(_D // 16):
                    outs[s][r, pl.ds(j * 16, 16)] = (
                        ins[s][r, pl.ds(j * 16, 16)] * _SCALE
                    )
                return carry

            lax.fori_loop(0, rows, scale_row, 0, unroll=2)

        for s in range(_NBUF):
            gather(s, s).start()
        for s in range(_NBUF):
            gather(s, s).wait()
            scale(s)
            start_stores(s, s)
            gather(s + _NBUF, s).start()

        def group(g, carry):
            for s in range(_NBUF):
                c = g * _NBUF + s
                gather(c, s).wait()
                wait_stores(c - _NBUF, s)
                scale(s)
                start_stores(c, s)
                gather(c + _NBUF, s).start()
            return carry

        lax.fori_loop(1, n_groups - 1, group, 0)

        for s in range(_NBUF):
            c = (n_groups - 1) * _NBUF + s
            gather(c, s).wait()
            wait_stores(c - _NBUF, s)
            scale(s)
            start_stores(c, s)
        for s in range(_NBUF):
            wait_stores(n_chunks - _NBUF + s, s)

    return lookup


@jax.jit
def kernel(x, emb_weight):
    b, l = x.shape
    idx = x.reshape(_NW, b // (_NW * _BPC), _BPC * l).astype(jnp.int32)
    return _make_lookup(b, l)(idx, emb_weight)


# speedup vs baseline: 1.9862x; 1.9862x over previous
import functools
import math

import jax
import jax.numpy as jnp
from jax import lax
from jax.experimental import pallas as pl
from jax.experimental.pallas import tpu as pltpu
from jax.experimental.pallas import tpu_sc as plsc

_D = 128
_SCALE = math.sqrt(float(_D))
_NC = 2
_NS = 16
_NW = _NC * _NS
_BPC = 2
_NBUF = 4


def _make_lookup(B: int, L: int):
    assert B % (_NW * _BPC * _NBUF) == 0
    b_per_w = B // _NW
    n_chunks = b_per_w // _BPC
    n_groups = n_chunks // _NBUF
    assert n_groups >= 3
    rows = _BPC * L
    mesh = plsc.VectorSubcoreMesh(
        core_axis_name="c", subcore_axis_name="s", num_cores=_NC, num_subcores=_NS
    )

    @functools.partial(
        pl.kernel,
        mesh=mesh,
        out_type=jax.ShapeDtypeStruct((B, L, _D), jnp.float32),
        scratch_types=(
            [pltpu.VMEM((n_chunks, rows), jnp.int32)]
            + [pltpu.VMEM((rows, _D), jnp.float32) for _ in range(2 * _NBUF)]
            + [pltpu.SemaphoreType.DMA for _ in range(2 * _NBUF)]
        ),
        compiler_params=pltpu.CompilerParams(use_tc_tiling_on_sc=True),
    )
    def lookup(idx_hbm, table_hbm, out_hbm, idx_v, *rest):
        ins = rest[:_NBUF]
        outs = rest[_NBUF:2 * _NBUF]
        gsems = rest[2 * _NBUF:3 * _NBUF]
        ssems = rest[3 * _NBUF:4 * _NBUF]
        wid = lax.axis_index("s") * _NC + lax.axis_index("c")
        base = wid * b_per_w
        pltpu.sync_copy(idx_hbm.at[wid], idx_v)

        def gather(c, s):
            return pltpu.make_async_copy(
                table_hbm.at[idx_v.at[c]], ins[s], gsems[s]
            )

        def stores(c, s):
            return [
                pltpu.make_async_copy(
                    outs[s].at[pl.ds(i * L, L)],
                    out_hbm.at[base + c * _BPC + i],
                    ssems[s],
                )
                for i in range(_BPC)
            ]

        def start_stores(c, s):
            for cp in stores(c, s):
                cp.start()

        def wait_stores(c, s):
            for cp in stores(c, s):
                cp.wait()

        def scale(s):
            @plsc.parallel_loop(0, rows, 1, unroll=4)
            def _scale_row(r):
                for j in range(_D // 16):
                    outs[s][r, pl.ds(j * 16, 16)] = (
                        ins[s][r, pl.ds(j * 16, 16)] * _SCALE
                    )

        for s in range(_NBUF):
            gather(s, s).start()
        for s in range(_NBUF):
            gather(s, s).wait()
            scale(s)
            start_stores(s, s)
            gather(s + _NBUF, s).start()

        def group(g, carry):
            for s in range(_NBUF):
                c = g * _NBUF + s
                gather(c, s).wait()
                wait_stores(c - _NBUF, s)
                scale(s)
                start_stores(c, s)
                gather(c + _NBUF, s).start()
            return carry

        lax.fori_loop(1, n_groups - 1, group, 0)

        for s in range(_NBUF):
            c = (n_groups - 1) * _NBUF + s
            gather(c, s).wait()
            wait_stores(c - _NBUF, s)
            scale(s)
            start_stores(c, s)
        for s in range(_NBUF):
            wait_stores(n_chunks - _NBUF + s, s)

    return lookup


@jax.jit
def kernel(x, emb_weight):
    b, l = x.shape
    idx = x.reshape(_NW, b // (_NW * _BPC), _BPC * l).astype(jnp.int32)
    return _make_lookup(b, l)(idx, emb_weight)
